# baseline (speedup 1.0000x reference)
import math

import jax
import jax.numpy as jnp
from jax import lax
from jax.experimental import pallas as pl
from jax.experimental.pallas import tpu as pltpu
from jax.experimental.pallas import tpu_sc as plsc

_VOCAB = 1000000
_D = 64
_PAD = 1
_B = 1024
_S = 200

_NC = 2
_NS = 16
_NW = _NC * _NS
_B_PER_W = _B // _NW

_TBLK = 2048
_NBLK = -(-_VOCAB // _TBLK)
_EROWS = _NBLK * 1024

_SA = 104
_SB = _S - _SA

_SCALE = math.sqrt(_D)


def _repack_body(in_ref, out_ref):
  blk = in_ref[...]
  out_ref[...] = jnp.concatenate(
      [blk[:, :1024].T, blk[:, 1024:].T], axis=1)


def _repack(embed_t):
  return pl.pallas_call(
      _repack_body,
      out_shape=jax.ShapeDtypeStruct((_EROWS, 128), jnp.float32),
      grid=(_NBLK,),
      in_specs=[pl.BlockSpec((_D, _TBLK), lambda j: (0, j))],
      out_specs=pl.BlockSpec((1024, 128), lambda j: (j, 0)),
  )(embed_t)


def _sc_body(tok_hbm, emb_hbm, pos_hbm,
             e_hbm, posr_hbm, mask_hbm,
             tok32, mask32, pid_v, eid_v, xid_a, xid_b, erows, prows, sem):
  cid = lax.axis_index("c")
  sid = lax.axis_index("s")
  wid = sid * _NC + cid
  b0 = wid * _B_PER_W
  iota = lax.iota(jnp.int32, 16)

  pltpu.sync_copy(tok_hbm.at[pl.ds(b0, _B_PER_W), :], tok32)

  def row_body(r, _):
    b = b0 + r

    def chunk(c, carry):
      off = pl.multiple_of(c * 16, 16)
      t = tok32[r, pl.ds(off, 16)]
      nonpad = jnp.where(t != _PAD, 1, 0).astype(jnp.int32)
      cs = plsc.cumsum(nonpad) + carry
      pid_v[pl.ds(off, 16)] = cs * nonpad + 1
      eid_v[pl.ds(off, 16)] = (
          lax.shift_left(lax.shift_right_logical(t, 11), 10)
          + (t & 1023))
      mask32[r, pl.ds(off, 16)] = 1 - nonpad
      xs = off + iota
      xid = xs * _B + b
      pl.when(off < _SA)(lambda: plsc.store_scatter(xid_a, [xs], xid))

      def _xb():
        plsc.store_scatter(xid_b, [xs - _SA], xid)

      pl.when(off >= _SA)(_xb)
      return carry + jnp.sum(nonpad)

    carry = lax.fori_loop(0, (_S - 8) // 16, chunk, jnp.int32(0))

    t = tok32[r, pl.ds(_S - 16, 16)]
    fresh = iota >= 8
    nonpad = jnp.where(t != _PAD, 1, 0).astype(jnp.int32)
    cs = plsc.cumsum(nonpad * fresh.astype(jnp.int32)) + carry
    xs = (_S - 16) + iota
    plsc.store_scatter(pid_v, [xs], cs * nonpad + 1, mask=fresh)
    eid_v[pl.ds(_S - 16, 16)] = (
        lax.shift_left(lax.shift_right_logical(t, 11), 10) + (t & 1023))
    plsc.store_scatter(mask32, [jnp.full((16,), r, jnp.int32), xs],
                       1 - nonpad, mask=fresh)
    plsc.store_scatter(xid_b, [xs - _SA], xs * _B + b, mask=fresh)

    h0 = pltpu.async_copy(emb_hbm.at[eid_v.at[pl.ds(0, _SA)]],
                          erows.at[pl.ds(0, _SA)], sem)
    h1 = pltpu.async_copy(emb_hbm.at[eid_v.at[pl.ds(_SA, _SB)]],
                          erows.at[pl.ds(_SA, _SB)], sem)
    h2 = pltpu.async_copy(pos_hbm.at[pid_v.at[pl.ds(0, _SA)]],
                          prows.at[pl.ds(0, _SA)], sem)
    h3 = pltpu.async_copy(pos_hbm.at[pid_v.at[pl.ds(_SA, _SB)]],
                          prows.at[pl.ds(_SA, _SB)], sem)
    h0.wait()
    h1.wait()
    h2.wait()
    h3.wait()

    pltpu.sync_copy(prows, posr_hbm.at[pl.ds(b * _S, _S), :])
    w0 = pltpu.async_copy(erows.at[pl.ds(0, _SA)], e_hbm.at[xid_a], sem)
    w1 = pltpu.async_copy(erows.at[pl.ds(_SA, _SB)], e_hbm.at[xid_b], sem)
    w0.wait()
    w1.wait()
    return 0

  lax.fori_loop(0, _B_PER_W, row_body, 0)
  pltpu.sync_copy(mask32, mask_hbm.at[pl.ds(b0, _B_PER_W), :])


def _sc_call(src_tokens, etab, pos_table):
  mesh = plsc.VectorSubcoreMesh(core_axis_name="c", subcore_axis_name="s")
  out_type = (
      jax.ShapeDtypeStruct((_S * _B, 128), jnp.float32),
      jax.ShapeDtypeStruct((_B * _S, _D), jnp.float32),
      jax.ShapeDtypeStruct((_B, _S), jnp.int32),
  )
  scratch = [
      pltpu.VMEM((_B_PER_W, _S), jnp.int32),
      pltpu.VMEM((_B_PER_W, _S), jnp.int32),
      pltpu.VMEM((_S + 8,), jnp.int32),
      pltpu.VMEM((_S + 8,), jnp.int32),
      pltpu.VMEM((_SA,), jnp.int32),
      pltpu.VMEM((_SB,), jnp.int32),
      pltpu.VMEM((_S, 128), jnp.float32),
      pltpu.VMEM((_S, _D), jnp.float32),
      pltpu.SemaphoreType.DMA,
  ]
  run = pl.kernel(
      _sc_body, mesh=mesh, out_type=out_type, scratch_types=scratch,
      compiler_params=pltpu.CompilerParams(
          use_tc_tiling_on_sc=False, needs_layout_passes=False))
  return run(src_tokens, etab, pos_table)


def _combine_body(e_ref, pos_ref, tok_ref, x_ref, ppos_ref):
  e = e_ref[...]
  pos = pos_ref[...]
  tok = tok_ref[...]
  par = (lax.shift_right_logical(tok, 10) & 1).T
  for k in range(8):
    ek = e[k]
    eh = jnp.where(par[:, k:k + 1] == 1, ek[:, _D:], ek[:, :_D])
    pk = pos[:, k * _D:(k + 1) * _D]
    x_ref[k, :, :] = eh * _SCALE + pk
    ppos_ref[:, k, :] = pk


def _combine(e3, pos2d, tok_t):
  return pl.pallas_call(
      _combine_body,
      out_shape=(
          jax.ShapeDtypeStruct((_S, _B, _D), jnp.float32),
          jax.ShapeDtypeStruct((_B, _S, _D), jnp.float32),
      ),
      grid=(_B // 128, _S // 8),
      in_specs=[
          pl.BlockSpec((8, 128, 128), lambda j, sb: (sb, j, 0)),
          pl.BlockSpec((128, 8 * _D), lambda j, sb: (j, sb)),
          pl.BlockSpec((8, 128), lambda j, sb: (sb, j)),
      ],
      out_specs=(
          pl.BlockSpec((8, 128, _D), lambda j, sb: (sb, j, 0)),
          pl.BlockSpec((128, 8, _D), lambda j, sb: (j, sb, 0)),
      ),
  )(e3, pos2d, tok_t)


@jax.jit
def _impl(src_tokens, embed_table, pos_table):
  etab = _repack(embed_table.T)
  e512, posr, mask_i32 = _sc_call(src_tokens, etab, pos_table)
  x, positions = _combine(
      e512.reshape(_S, _B, 128), posr.reshape(_B, _S * _D), src_tokens.T)
  return x, mask_i32.astype(jnp.bool_), positions


def kernel(src_tokens, src_lengths, embed_table, pos_table):
  del src_lengths
  return _impl(src_tokens.astype(jnp.int32), embed_table, pos_table)


# speedup vs baseline: 1.2779x; 1.2779x over previous
import math

import jax
import jax.numpy as jnp
from jax import lax
from jax.experimental import pallas as pl
from jax.experimental.pallas import tpu as pltpu
from jax.experimental.pallas import tpu_sc as plsc

_VOCAB = 1000000
_D = 64
_PAD = 1
_B = 1024
_S = 200

_NC = 2
_NS = 16
_NW = _NC * _NS
_B_PER_W = _B // _NW

_TBLK = 2048
_NBLK = -(-_VOCAB // _TBLK)
_EROWS = _NBLK * _TBLK

_SA = 104
_SB = _S - _SA

_SCALE = math.sqrt(_D)


def _repack_body(in_ref, out_ref):
  blk = in_ref[...]
  out_ref[:, 0:_D] = blk.T


def _repack(embed_t):
  return pl.pallas_call(
      _repack_body,
      out_shape=jax.ShapeDtypeStruct((_EROWS, 128), jnp.float32),
      grid=(_NBLK,),
      in_specs=[pl.BlockSpec((_D, _TBLK), lambda j: (0, j))],
      out_specs=pl.BlockSpec((_TBLK, 128), lambda j: (j, 0)),
  )(embed_t)


def _sc_body(tok_hbm, emb_hbm, pos_hbm,
             e_hbm, posr_hbm, mask_hbm,
             tok32, mask32, pid_v, xid_a, xid_b, erows, prows, sem):
  cid = lax.axis_index("c")
  sid = lax.axis_index("s")
  wid = sid * _NC + cid
  b0 = wid * _B_PER_W
  iota = lax.iota(jnp.int32, 16)

  pltpu.sync_copy(tok_hbm.at[pl.ds(b0, _B_PER_W), :], tok32)

  def row_body(r, _):
    b = b0 + r

    def chunk(c, carry):
      off = pl.multiple_of(c * 16, 16)
      t = tok32[r, pl.ds(off, 16)]
      nonpad = jnp.where(t != _PAD, 1, 0).astype(jnp.int32)
      cs = plsc.cumsum(nonpad) + carry
      pid_v[pl.ds(off, 16)] = cs * nonpad + 1
      mask32[r, pl.ds(off, 16)] = 1 - nonpad
      xs = off + iota
      xid = xs * _B + b
      pl.when(off < _SA)(lambda: plsc.store_scatter(xid_a, [xs], xid))

      def _xb():
        plsc.store_scatter(xid_b, [xs - _SA], xid)

      pl.when(off >= _SA)(_xb)
      return carry + jnp.sum(nonpad)

    carry = lax.fori_loop(0, (_S - 8) // 16, chunk, jnp.int32(0))

    t = tok32[r, pl.ds(_S - 16, 16)]
    fresh = iota >= 8
    nonpad = jnp.where(t != _PAD, 1, 0).astype(jnp.int32)
    cs = plsc.cumsum(nonpad * fresh.astype(jnp.int32)) + carry
    xs = (_S - 16) + iota
    plsc.store_scatter(pid_v, [xs], cs * nonpad + 1, mask=fresh)
    plsc.store_scatter(mask32, [jnp.full((16,), r, jnp.int32), xs],
                       1 - nonpad, mask=fresh)
    plsc.store_scatter(xid_b, [xs - _SA], xs * _B + b, mask=fresh)

    h0 = pltpu.async_copy(emb_hbm.at[tok32.at[r, pl.ds(0, _SA)]],
                          erows.at[pl.ds(0, _SA)], sem)
    h1 = pltpu.async_copy(emb_hbm.at[tok32.at[r, pl.ds(_SA, _SB)]],
                          erows.at[pl.ds(_SA, _SB)], sem)
    h2 = pltpu.async_copy(pos_hbm.at[pid_v.at[pl.ds(0, _SA)]],
                          prows.at[pl.ds(0, _SA)], sem)
    h3 = pltpu.async_copy(pos_hbm.at[pid_v.at[pl.ds(_SA, _SB)]],
                          prows.at[pl.ds(_SA, _SB)], sem)
    h0.wait()
    h1.wait()
    h2.wait()
    h3.wait()

    pltpu.sync_copy(prows, posr_hbm.at[pl.ds(b * _S, _S), :])
    w0 = pltpu.async_copy(erows.at[pl.ds(0, _SA)], e_hbm.at[xid_a], sem)
    w1 = pltpu.async_copy(erows.at[pl.ds(_SA, _SB)], e_hbm.at[xid_b], sem)
    w0.wait()
    w1.wait()
    return 0

  lax.fori_loop(0, _B_PER_W, row_body, 0)
  pltpu.sync_copy(mask32, mask_hbm.at[pl.ds(b0, _B_PER_W), :])


def _sc_call(src_tokens, etab, pos128):
  mesh = plsc.VectorSubcoreMesh(core_axis_name="c", subcore_axis_name="s")
  out_type = (
      jax.ShapeDtypeStruct((_S * _B, 128), jnp.float32),
      jax.ShapeDtypeStruct((_B * _S, 128), jnp.float32),
      jax.ShapeDtypeStruct((_B, _S), jnp.int32),
  )
  scratch = [
      pltpu.VMEM((_B_PER_W, _S), jnp.int32),
      pltpu.VMEM((_B_PER_W, _S), jnp.int32),
      pltpu.VMEM((_S + 8,), jnp.int32),
      pltpu.VMEM((_SA,), jnp.int32),
      pltpu.VMEM((_SB,), jnp.int32),
      pltpu.VMEM((_S, 128), jnp.float32),
      pltpu.VMEM((_S, 128), jnp.float32),
      pltpu.SemaphoreType.DMA,
  ]
  run = pl.kernel(
      _sc_body, mesh=mesh, out_type=out_type, scratch_types=scratch,
      compiler_params=pltpu.CompilerParams(
          use_tc_tiling_on_sc=False, needs_layout_passes=False))
  return run(src_tokens, etab, pos128)


def _combine_body(e_ref, pos_ref, x_ref, ppos_ref):
  e = e_ref[...]
  pos = pos_ref[...]
  for k in range(8):
    pk = pos[:, k, 0:_D]
    xk = e[k][:, 0:_D] * _SCALE + pk
    x_ref[k] = xk.T
    ppos_ref[k] = pk.T


def _combine(e3, pos3):
  return pl.pallas_call(
      _combine_body,
      out_shape=(
          jax.ShapeDtypeStruct((_S, _D, _B), jnp.float32),
          jax.ShapeDtypeStruct((_S, _D, _B), jnp.float32),
      ),
      grid=(_B // 128, _S // 8),
      in_specs=[
          pl.BlockSpec((8, 128, 128), lambda j, sb: (sb, j, 0)),
          pl.BlockSpec((128, 8, 128), lambda j, sb: (j, sb, 0)),
      ],
      out_specs=(
          pl.BlockSpec((8, _D, 128), lambda j, sb: (sb, 0, j)),
          pl.BlockSpec((8, _D, 128), lambda j, sb: (sb, 0, j)),
      ),
  )(e3, pos3)


@jax.jit
def _impl(src_tokens, embed_table, pos_table):
  etab = _repack(embed_table.T)
  pos128 = jnp.concatenate([pos_table, pos_table], axis=1)
  e512, posr, mask_i32 = _sc_call(src_tokens, etab, pos128)
  xt, ppt = _combine(
      e512.reshape(_S, _B, 128), posr.reshape(_B, _S, 128))
  x = jnp.transpose(xt, (0, 2, 1))
  positions = jnp.transpose(ppt, (2, 0, 1))
  return x, mask_i32.astype(jnp.bool_), positions


def kernel(src_tokens, src_lengths, embed_table, pos_table):
  del src_lengths
  return _impl(src_tokens.astype(jnp.int32), embed_table, pos_table)
